# baseline (speedup 1.0000x reference)
import functools

import jax
import jax.numpy as jnp
from jax import lax
from jax.experimental import pallas as pl
from jax.experimental.pallas import tpu as pltpu
from jax.experimental.pallas import tpu_sc as plsc

D = 1024
S = 4096
R = 2 * S
V = 199
VP = 256
NW = 32
HALF_PER_W = S // NW
STG = VP // 16
BATCH = 16
NB = HALF_PER_W // BATCH


@functools.partial(
    pl.kernel,
    mesh=plsc.VectorSubcoreMesh(core_axis_name="c", subcore_axis_name="s"),
    out_type=jax.ShapeDtypeStruct((R * D,), jnp.float32),
    scratch_types=[
        pltpu.VMEM((HALF_PER_W,), jnp.int32),
        pltpu.VMEM_SHARED((VP * D,), jnp.float32),
        pltpu.VMEM_SHARED((VP * D,), jnp.float32),
        pltpu.SemaphoreType.DMA,
        pltpu.SemaphoreType.DMA,
        pltpu.SemaphoreType.DMA,
    ],
)
def _gather_kernel(idx_hbm, text_hbm, image_hbm, out_hbm, idx_v,
                   text_sp, image_sp, sem_a, sem_b, sem_c):
    sid = lax.axis_index("s")
    wid = sid * 2 + lax.axis_index("c")
    base = wid * HALF_PER_W

    stg0 = sid * (STG * D)
    st_t = pltpu.async_copy(
        text_hbm.at[pl.ds(stg0, STG * D)], text_sp.at[pl.ds(stg0, STG * D)], sem_a)
    st_i = pltpu.async_copy(
        image_hbm.at[pl.ds(stg0, STG * D)], image_sp.at[pl.ds(stg0, STG * D)], sem_b)
    pltpu.sync_copy(idx_hbm.at[pl.ds(base, HALF_PER_W)], idx_v)
    st_t.wait()
    st_i.wait()
    plsc.subcore_barrier()

    sems = (sem_a, sem_b, sem_c)

    def fire_batch(b):
        sem = sems[b % 3]
        iv = idx_v[pl.ds(b * BATCH, BATCH)]
        handles = []
        for l in range(BATCH):
            k = b * BATCH + l
            off = iv[l] * D
            handles.append(pltpu.async_copy(
                text_sp.at[pl.ds(off, D)],
                out_hbm.at[pl.ds((base + k) * D, D)], sem))
            handles.append(pltpu.async_copy(
                image_sp.at[pl.ds(off, D)],
                out_hbm.at[pl.ds((S + base + k) * D, D)], sem))
        return handles

    pending = [None] * NB
    pending[0] = fire_batch(0)
    if NB > 1:
        pending[1] = fire_batch(1)
    for b in range(NB):
        for h in pending[b]:
            h.wait()
        if b + 2 < NB:
            pending[b + 2] = fire_batch(b + 2)


def kernel(hidden_states, text_end_position, multiway_split_position, text_table, image_table):
    del hidden_states, multiway_split_position
    idx = text_end_position.reshape(S).astype(jnp.int32)
    text_p = jnp.pad(text_table, ((0, VP - V), (0, 0))).reshape(VP * D)
    image_p = jnp.pad(image_table[:V], ((0, VP - V), (0, 0))).reshape(VP * D)
    out = _gather_kernel(idx, text_p, image_p)
    return out.reshape(1, R, D)


# speedup vs baseline: 1.6998x; 1.6998x over previous
import functools

import jax
import jax.numpy as jnp
from jax import lax
from jax.experimental import pallas as pl
from jax.experimental.pallas import tpu as pltpu
from jax.experimental.pallas import tpu_sc as plsc

D = 1024
S = 4096
R = 2 * S
V = 199
VP = 256
NW = 32
HALF_PER_W = S // NW
CH = 32
NT = HALF_PER_W // CH
TB = 512
NTB = S // TB


@functools.partial(
    pl.kernel,
    mesh=plsc.VectorSubcoreMesh(core_axis_name="c", subcore_axis_name="s"),
    out_type=jax.ShapeDtypeStruct((R, D), jnp.float32),
    scratch_types=[
        pltpu.VMEM((HALF_PER_W,), jnp.int32),
        pltpu.VMEM((CH, D), jnp.float32),
        pltpu.VMEM((CH, D), jnp.float32),
        pltpu.SemaphoreType.DMA,
        pltpu.SemaphoreType.DMA,
        pltpu.SemaphoreType.DMA,
        pltpu.SemaphoreType.DMA,
    ],
)
def _sc_image_half(idx_hbm, image_hbm, out_hbm, idx_v,
                   buf_a, buf_b, sg_a, sg_b, ss_a, ss_b):
    wid = lax.axis_index("s") * 2 + lax.axis_index("c")
    base = wid * HALF_PER_W
    pltpu.sync_copy(idx_hbm.at[pl.ds(base, HALF_PER_W)], idx_v)

    bufs = (buf_a, buf_b)
    sg = (sg_a, sg_b)
    ss = (ss_a, ss_b)

    def start_gather(i):
        b = i % 2
        idx_slice = idx_v.at[pl.ds(i * CH, CH)]
        return pltpu.async_copy(image_hbm.at[idx_slice], bufs[b], sg[b])

    def start_store(i):
        b = i % 2
        dst = out_hbm.at[pl.ds(S + base + i * CH, CH)]
        return pltpu.async_copy(bufs[b], dst, ss[b])

    g = [None] * NT
    s = [None] * NT
    g[0] = start_gather(0)
    for i in range(NT):
        if i + 1 < NT:
            if i >= 1:
                s[i - 1].wait()
            g[i + 1] = start_gather(i + 1)
        g[i].wait()
        s[i] = start_store(i)
    s[NT - 2].wait()
    s[NT - 1].wait()


def _tc_body(idx_ref, tab_ref, _aliased_ref, out_ref):
    idx_b = idx_ref[0, 0, :]
    cols = lax.broadcasted_iota(jnp.int32, (TB, VP), 1)
    one_hot = (idx_b[:, None] == cols).astype(jnp.float32)
    out_ref[...] = jnp.dot(one_hot, tab_ref[...],
                           preferred_element_type=jnp.float32)


_tc_text_half = pl.pallas_call(
    _tc_body,
    grid=(NTB,),
    in_specs=[
        pl.BlockSpec((1, 1, TB), lambda i: (i, 0, 0)),
        pl.BlockSpec((VP, D), lambda i: (0, 0)),
        pl.BlockSpec(memory_space=pl.ANY),
    ],
    out_specs=pl.BlockSpec((TB, D), lambda i: (i, 0)),
    out_shape=jax.ShapeDtypeStruct((R, D), jnp.float32),
    input_output_aliases={2: 0},
)


def kernel(hidden_states, text_end_position, multiway_split_position, text_table, image_table):
    del hidden_states, multiway_split_position
    idx = text_end_position.reshape(S).astype(jnp.int32)
    part = _sc_image_half(idx, image_table)
    text_p = jnp.pad(text_table, ((0, VP - V), (0, 0)))
    out = _tc_text_half(idx.reshape(NTB, 1, TB), text_p, part)
    return out.reshape(1, R, D)


# speedup vs baseline: 1.7113x; 1.0068x over previous
import functools

import jax
import jax.numpy as jnp
from jax import lax
from jax.experimental import pallas as pl
from jax.experimental.pallas import tpu as pltpu
from jax.experimental.pallas import tpu_sc as plsc

D = 1024
S = 4096
R = 2 * S
V = 199
VP = 256
NW = 32
HALF_PER_W = S // NW
CH = 32
NT = HALF_PER_W // CH
TB = 512
NTB = S // TB


@functools.partial(
    pl.kernel,
    mesh=plsc.VectorSubcoreMesh(core_axis_name="c", subcore_axis_name="s"),
    out_type=jax.ShapeDtypeStruct((R, D), jnp.float32),
    scratch_types=[
        pltpu.VMEM((HALF_PER_W,), jnp.int32),
        pltpu.VMEM((CH, D), jnp.float32),
        pltpu.VMEM((CH, D), jnp.float32),
        pltpu.SemaphoreType.DMA,
        pltpu.SemaphoreType.DMA,
        pltpu.SemaphoreType.DMA,
        pltpu.SemaphoreType.DMA,
    ],
)
def _sc_image_half(idx_hbm, image_hbm, out_hbm, idx_v,
                   buf_a, buf_b, sg_a, sg_b, ss_a, ss_b):
    wid = lax.axis_index("s") * 2 + lax.axis_index("c")
    base = wid * HALF_PER_W
    pltpu.sync_copy(idx_hbm.at[pl.ds(base, HALF_PER_W)], idx_v)

    bufs = (buf_a, buf_b)
    sg = (sg_a, sg_b)
    ss = (ss_a, ss_b)

    def start_gather(i):
        b = i % 2
        idx_slice = idx_v.at[pl.ds(i * CH, CH)]
        return pltpu.async_copy(image_hbm.at[idx_slice], bufs[b], sg[b])

    def start_store(i):
        b = i % 2
        dst = out_hbm.at[pl.ds(S + base + i * CH, CH)]
        return pltpu.async_copy(bufs[b], dst, ss[b])

    g = [None] * NT
    s = [None] * NT
    g[0] = start_gather(0)
    for i in range(NT):
        if i + 1 < NT:
            if i >= 1:
                s[i - 1].wait()
            g[i + 1] = start_gather(i + 1)
        g[i].wait()
        s[i] = start_store(i)
    s[NT - 2].wait()
    s[NT - 1].wait()


def _tc_body(idx_ref, tab_ref, _aliased_ref, out_ref):
    idx_b = idx_ref[0, 0, :]
    cols = lax.broadcasted_iota(jnp.int32, (TB, VP), 1)
    one_hot = (idx_b[:, None] == cols).astype(jnp.bfloat16)
    out_ref[...] = jnp.dot(one_hot, tab_ref[...],
                           preferred_element_type=jnp.float32)


_tc_text_half = pl.pallas_call(
    _tc_body,
    grid=(NTB,),
    in_specs=[
        pl.BlockSpec((1, 1, TB), lambda i: (i, 0, 0)),
        pl.BlockSpec((VP, D), lambda i: (0, 0)),
        pl.BlockSpec(memory_space=pl.ANY),
    ],
    out_specs=pl.BlockSpec((TB, D), lambda i: (i, 0)),
    out_shape=jax.ShapeDtypeStruct((R, D), jnp.float32),
    input_output_aliases={2: 0},
)


def kernel(hidden_states, text_end_position, multiway_split_position, text_table, image_table):
    del hidden_states, multiway_split_position
    idx = text_end_position.reshape(S).astype(jnp.int32)
    part = _sc_image_half(idx, image_table)
    text_p = jnp.pad(text_table, ((0, VP - V), (0, 0))).astype(jnp.bfloat16)
    out = _tc_text_half(idx.reshape(NTB, 1, TB), text_p, part)
    return out.reshape(1, R, D)


# speedup vs baseline: 1.7184x; 1.0041x over previous
import functools

import jax
import jax.numpy as jnp
from jax import lax
from jax.experimental import pallas as pl
from jax.experimental.pallas import tpu as pltpu
from jax.experimental.pallas import tpu_sc as plsc

D = 1024
S = 4096
R = 2 * S
V = 199
VP = 256
NW = 32

SC_SKIP = 1024
SC_ROWS = S - SC_SKIP
ROWS_PER_W = SC_ROWS // NW
CH = 32
NT = ROWS_PER_W // CH

TB = 512
NTB = (S + SC_SKIP) // TB
NTXT = S // TB


@functools.partial(
    pl.kernel,
    mesh=plsc.VectorSubcoreMesh(core_axis_name="c", subcore_axis_name="s"),
    out_type=jax.ShapeDtypeStruct((R, D), jnp.float32),
    scratch_types=[
        pltpu.VMEM((ROWS_PER_W,), jnp.int32),
        pltpu.VMEM((CH, D), jnp.float32),
        pltpu.VMEM((CH, D), jnp.float32),
        pltpu.SemaphoreType.DMA,
        pltpu.SemaphoreType.DMA,
        pltpu.SemaphoreType.DMA,
        pltpu.SemaphoreType.DMA,
    ],
)
def _sc_image_part(idx_hbm, image_hbm, out_hbm, idx_v,
                   buf_a, buf_b, sg_a, sg_b, ss_a, ss_b):
    wid = lax.axis_index("s") * 2 + lax.axis_index("c")
    base = SC_SKIP + wid * ROWS_PER_W
    pltpu.sync_copy(idx_hbm.at[pl.ds(base, ROWS_PER_W)], idx_v)

    bufs = (buf_a, buf_b)
    sg = (sg_a, sg_b)
    ss = (ss_a, ss_b)

    def start_gather(i):
        b = i % 2
        idx_slice = idx_v.at[pl.ds(i * CH, CH)]
        return pltpu.async_copy(image_hbm.at[idx_slice], bufs[b], sg[b])

    def start_store(i):
        b = i % 2
        dst = out_hbm.at[pl.ds(S + base + i * CH, CH)]
        return pltpu.async_copy(bufs[b], dst, ss[b])

    g = [None] * NT
    s = [None] * NT
    g[0] = start_gather(0)
    for i in range(NT):
        if i + 1 < NT:
            if i >= 1:
                s[i - 1].wait()
            g[i + 1] = start_gather(i + 1)
        g[i].wait()
        s[i] = start_store(i)
    s[NT - 2].wait()
    s[NT - 1].wait()


def _tc_body(idx_ref, ttab_ref, itab_ref, _aliased_ref, out_ref):
    i = pl.program_id(0)
    idx_b = idx_ref[0, 0, :]
    cols = lax.broadcasted_iota(jnp.int32, (TB, VP), 1)
    one_hot = (idx_b[:, None] == cols).astype(jnp.bfloat16)
    rows_t = jnp.dot(one_hot, ttab_ref[...], preferred_element_type=jnp.float32)
    rows_i = jnp.dot(one_hot, itab_ref[...], preferred_element_type=jnp.float32)
    out_ref[...] = jnp.where(i < NTXT, rows_t, rows_i)


_tc_part = pl.pallas_call(
    _tc_body,
    grid=(NTB,),
    in_specs=[
        pl.BlockSpec((1, 1, TB), lambda i: (i, 0, 0)),
        pl.BlockSpec((VP, D), lambda i: (0, 0)),
        pl.BlockSpec((VP, D), lambda i: (0, 0)),
        pl.BlockSpec(memory_space=pl.ANY),
    ],
    out_specs=pl.BlockSpec((TB, D), lambda i: (i, 0)),
    out_shape=jax.ShapeDtypeStruct((R, D), jnp.float32),
    input_output_aliases={3: 0},
)


def kernel(hidden_states, text_end_position, multiway_split_position, text_table, image_table):
    del hidden_states, multiway_split_position
    idx = text_end_position.reshape(S).astype(jnp.int32)
    part = _sc_image_part(idx, image_table)
    text_p = jnp.pad(text_table, ((0, VP - V), (0, 0))).astype(jnp.bfloat16)
    image_p = jnp.pad(image_table[:V], ((0, VP - V), (0, 0))).astype(jnp.bfloat16)
    idx_tc = jnp.concatenate([idx, idx[:SC_SKIP]]).reshape(NTB, 1, TB)
    out = _tc_part(idx_tc, text_p, image_p, part)
    return out.reshape(1, R, D)


# speedup vs baseline: 1.7786x; 1.0350x over previous
import functools

import jax
import jax.numpy as jnp
from jax import lax
from jax.experimental import pallas as pl
from jax.experimental.pallas import tpu as pltpu
from jax.experimental.pallas import tpu_sc as plsc

D = 1024
S = 4096
R = 2 * S
V = 199
VP = 256
NW = 32

SC_SKIP = 1024
SC_ROWS = S - SC_SKIP
ROWS_PER_W = SC_ROWS // NW
CH = 32
NT = ROWS_PER_W // CH

TB = 1024
NTB = (S + SC_SKIP) // TB
VC = 2 * VP


@functools.partial(
    pl.kernel,
    mesh=plsc.VectorSubcoreMesh(core_axis_name="c", subcore_axis_name="s"),
    out_type=jax.ShapeDtypeStruct((R, D), jnp.float32),
    scratch_types=[
        pltpu.VMEM((ROWS_PER_W,), jnp.int32),
        pltpu.VMEM((CH, D), jnp.float32),
        pltpu.VMEM((CH, D), jnp.float32),
        pltpu.SemaphoreType.DMA,
        pltpu.SemaphoreType.DMA,
        pltpu.SemaphoreType.DMA,
        pltpu.SemaphoreType.DMA,
    ],
)
def _sc_image_part(idx_hbm, image_hbm, out_hbm, idx_v,
                   buf_a, buf_b, sg_a, sg_b, ss_a, ss_b):
    wid = lax.axis_index("s") * 2 + lax.axis_index("c")
    base = SC_SKIP + wid * ROWS_PER_W
    pltpu.sync_copy(idx_hbm.at[pl.ds(base, ROWS_PER_W)], idx_v)

    bufs = (buf_a, buf_b)
    sg = (sg_a, sg_b)
    ss = (ss_a, ss_b)

    def start_gather(i):
        b = i % 2
        idx_slice = idx_v.at[pl.ds(i * CH, CH)]
        return pltpu.async_copy(image_hbm.at[idx_slice], bufs[b], sg[b])

    def start_store(i):
        b = i % 2
        dst = out_hbm.at[pl.ds(S + base + i * CH, CH)]
        return pltpu.async_copy(bufs[b], dst, ss[b])

    g = [None] * NT
    s = [None] * NT
    g[0] = start_gather(0)
    for i in range(NT):
        if i + 1 < NT:
            if i >= 1:
                s[i - 1].wait()
            g[i + 1] = start_gather(i + 1)
        g[i].wait()
        s[i] = start_store(i)
    s[NT - 2].wait()
    s[NT - 1].wait()


def _tc_body(idx_ref, tab_ref, _aliased_ref, out_ref):
    idx_b = idx_ref[0, 0, :]
    cols = lax.broadcasted_iota(jnp.int32, (TB, VC), 1)
    one_hot = (idx_b[:, None] == cols).astype(jnp.bfloat16)
    out_ref[...] = jnp.dot(one_hot, tab_ref[...],
                           preferred_element_type=jnp.float32)


_tc_part = pl.pallas_call(
    _tc_body,
    grid=(NTB,),
    in_specs=[
        pl.BlockSpec((1, 1, TB), lambda i: (i, 0, 0)),
        pl.BlockSpec((VC, D), lambda i: (0, 0)),
        pl.BlockSpec(memory_space=pl.ANY),
    ],
    out_specs=pl.BlockSpec((TB, D), lambda i: (i, 0)),
    out_shape=jax.ShapeDtypeStruct((R, D), jnp.float32),
    input_output_aliases={2: 0},
)


def kernel(hidden_states, text_end_position, multiway_split_position, text_table, image_table):
    del hidden_states, multiway_split_position
    idx = text_end_position.reshape(S).astype(jnp.int32)
    part = _sc_image_part(idx, image_table)
    tabs = jnp.concatenate([
        jnp.pad(text_table, ((0, VP - V), (0, 0))),
        jnp.pad(image_table[:V], ((0, VP - V), (0, 0))),
    ]).astype(jnp.bfloat16)
    idx_tc = jnp.concatenate([idx, idx[:SC_SKIP] + VP]).reshape(NTB, 1, TB)
    out = _tc_part(idx_tc, tabs, part)
    return out.reshape(1, R, D)
